# baseline (speedup 1.0000x reference)
import functools

import jax
import jax.numpy as jnp
from jax import lax
from jax.experimental import pallas as pl
from jax.experimental.pallas import tpu as pltpu
from jax.experimental.pallas import tpu_sc as plsc

_N = 10000
_D = 128
_E = 320000
_NC = 2
_NS = 16
_NW = _NC * _NS
_K = 128
_EPT = _E // _NW
_CH = -(-_EPT // _K)
_EPTP = _CH * _K
_NH = 10240
_NPT = 632
_NPL = _N - (_NS - 1) * _NPT
_BM = 512
_BA = 400


def _gemm_body(x_ref, w_ref, o_ref):
    o_ref[...] = lax.dot_general(
        x_ref[...], w_ref[...], (((1,), (1,)), ((), ())),
        preferred_element_type=jnp.float32)


def _add_body(a_ref, b_ref, o_ref):
    o_ref[...] = a_ref[...] + b_ref[...]


def _seg_body(src_hbm, dst_hbm, h_hbm, z_hbm, out_hbm,
              src_idx, dst_buf, rows, gsems, dsems, acc):
    c = lax.axis_index("c")
    s = lax.axis_index("s")
    w = c * _NS + s
    pltpu.sync_copy(src_hbm.at[w], src_idx)
    @pl.when(s < _NS - 1)
    def _():
        pltpu.sync_copy(z_hbm.at[pl.ds(s * _NPT, _NPT)],
                        acc.at[pl.ds(s * _NPT, _NPT)])

    @pl.when(s == _NS - 1)
    def _():
        pltpu.sync_copy(z_hbm.at[pl.ds(s * _NPT, _NPL)],
                        acc.at[pl.ds(s * _NPT, _NPL)])

    plsc.subcore_barrier()

    def fire(j, b):
        pltpu.async_copy(dst_hbm.at[w, j], dst_buf.at[b], dsems.at[b])
        pltpu.async_copy(h_hbm.at[src_idx.at[pl.ds(j * _K, _K)]],
                         rows.at[b], gsems.at[b])

    fire(0, 0)
    fire(1, 1)

    def chunk(j, carry):
        b = lax.rem(j, 2)
        pltpu.make_async_copy(dst_hbm.at[w, j], dst_buf.at[b],
                              dsems.at[b]).wait()
        pltpu.make_async_copy(h_hbm.at[src_idx.at[pl.ds(j * _K, _K)]],
                              rows.at[b], gsems.at[b]).wait()
        pltpu.sync_copy(rows.at[b], acc.at[dst_buf.at[b]], add=True)

        @pl.when(j + 2 < _CH)
        def _():
            fire(j + 2, b)
        return carry

    lax.fori_loop(0, _CH, chunk, 0)

    plsc.subcore_barrier()

    @pl.when(s < _NS - 1)
    def _():
        pltpu.sync_copy(acc.at[pl.ds(s * _NPT, _NPT)],
                        out_hbm.at[c, pl.ds(s * _NPT, _NPT)])

    @pl.when(s == _NS - 1)
    def _():
        pltpu.sync_copy(acc.at[pl.ds(s * _NPT, _NPL)],
                        out_hbm.at[c, pl.ds(s * _NPT, _NPL)])


def kernel(x, edge_index, weight):
    n, d = x.shape

    x_pad = jnp.pad(x, ((0, _NH - n), (0, 0)))
    h = pl.pallas_call(
        _gemm_body,
        grid=(_NH // _BM,),
        in_specs=[pl.BlockSpec((_BM, d), lambda i: (i, 0)),
                  pl.BlockSpec(weight.shape, lambda i: (0, 0))],
        out_specs=pl.BlockSpec((_BM, d), lambda i: (i, 0)),
        out_shape=jax.ShapeDtypeStruct((_NH, d), jnp.float32),
    )(x_pad, weight)

    src = jnp.pad(edge_index[0].reshape(_NW, _EPT),
                  ((0, 0), (0, _EPTP - _EPT)), constant_values=n)
    dst = jnp.pad(edge_index[1].reshape(_NW, _EPT),
                  ((0, 0), (0, _EPTP - _EPT)),
                  constant_values=0).reshape(_NW, _CH, _K)
    zeros = jnp.zeros((n, d), jnp.float32)

    mesh = plsc.VectorSubcoreMesh(core_axis_name="c", subcore_axis_name="s")
    seg = pl.kernel(
        _seg_body,
        out_type=jax.ShapeDtypeStruct((_NC, n, d), jnp.float32),
        mesh=mesh,
        scratch_types=[
            pltpu.VMEM((_EPTP,), jnp.int32),
            pltpu.VMEM((2, _K), jnp.int32),
            pltpu.VMEM((2, _K, _D), jnp.float32),
            pltpu.SemaphoreType.DMA((2,)),
            pltpu.SemaphoreType.DMA((2,)),
            pltpu.VMEM_SHARED((_N, _D), jnp.float32),
        ],
    )
    parts = seg(src, dst, h, zeros)

    out = pl.pallas_call(
        _add_body,
        grid=(n // _BA,),
        in_specs=[pl.BlockSpec((_BA, d), lambda i: (i, 0)),
                  pl.BlockSpec((_BA, d), lambda i: (i, 0))],
        out_specs=pl.BlockSpec((_BA, d), lambda i: (i, 0)),
        out_shape=jax.ShapeDtypeStruct((n, d), jnp.float32),
    )(parts[0], parts[1])
    return out


# speedup vs baseline: 1.2441x; 1.2441x over previous
import functools

import jax
import jax.numpy as jnp
from jax import lax
from jax.experimental import pallas as pl
from jax.experimental.pallas import tpu as pltpu
from jax.experimental.pallas import tpu_sc as plsc

_N = 10000
_D = 128
_E = 320000
_NC = 2
_NS = 16
_NW = _NC * _NS
_EPT = _E // _NW
_K = 125
_CH = _EPT // _K
_NP = 10240
_NPT = _NP // _NS
_BM = 400


def _gemm_body(x_ref, w_ref, o_ref):
    o_ref[...] = lax.dot_general(
        x_ref[...], w_ref[...], (((1,), (1,)), ((), ())),
        preferred_element_type=jnp.float32)


def _add_body(a_ref, b_ref, o_ref):
    o_ref[...] = a_ref[...] + b_ref[...]


def _seg_body(src_hbm, dst_hbm, h_hbm, z_hbm, out_hbm,
              src_idx, dst_idx, rows, sem, acc):
    c = lax.axis_index("c")
    s = lax.axis_index("s")
    w = c * _NS + s
    pltpu.sync_copy(src_hbm.at[w], src_idx)
    pltpu.sync_copy(dst_hbm.at[w], dst_idx)
    pltpu.sync_copy(z_hbm.at[pl.ds(s * _NPT, _NPT)],
                    acc.at[pl.ds(s * _NPT, _NPT)])
    plsc.subcore_barrier()

    def chunk(j, carry):
        pltpu.async_copy(h_hbm.at[src_idx.at[j]], rows, sem).wait()
        pltpu.sync_copy(rows, acc.at[dst_idx.at[j]], add=True)
        return carry

    lax.fori_loop(0, _CH, chunk, 0)

    plsc.subcore_barrier()
    pltpu.sync_copy(acc.at[pl.ds(s * _NPT, _NPT)],
                    out_hbm.at[c, pl.ds(s * _NPT, _NPT)])


def kernel(x, edge_index, weight):
    n, d = x.shape

    h = pl.pallas_call(
        _gemm_body,
        grid=(n // _BM,),
        in_specs=[pl.BlockSpec((_BM, d), lambda i: (i, 0)),
                  pl.BlockSpec(weight.shape, lambda i: (0, 0))],
        out_specs=pl.BlockSpec((_BM, d), lambda i: (i, 0)),
        out_shape=jax.ShapeDtypeStruct((n, d), jnp.float32),
    )(x, weight)

    src = edge_index[0].reshape(_NW, _CH, _K)
    dst = edge_index[1].reshape(_NW, _CH, _K)
    zeros = jnp.zeros((_NP, d), jnp.float32)

    mesh = plsc.VectorSubcoreMesh(core_axis_name="c", subcore_axis_name="s")
    seg = pl.kernel(
        _seg_body,
        out_type=jax.ShapeDtypeStruct((_NC, _NP, d), jnp.float32),
        mesh=mesh,
        scratch_types=[
            pltpu.VMEM((_CH, _K), jnp.int32),
            pltpu.VMEM((_CH, _K), jnp.int32),
            pltpu.VMEM((_K, _D), jnp.float32),
            pltpu.SemaphoreType.DMA,
            pltpu.VMEM_SHARED((_NP, _D), jnp.float32),
        ],
    )
    parts = seg(src, dst, h, zeros)

    out = pl.pallas_call(
        _add_body,
        grid=(n // _BM,),
        in_specs=[pl.BlockSpec((_BM, d), lambda i: (i, 0)),
                  pl.BlockSpec((_BM, d), lambda i: (i, 0))],
        out_specs=pl.BlockSpec((_BM, d), lambda i: (i, 0)),
        out_shape=jax.ShapeDtypeStruct((n, d), jnp.float32),
    )(parts[0], parts[1])
    return out


# speedup vs baseline: 1.6282x; 1.3088x over previous
import functools

import jax
import jax.numpy as jnp
from jax import lax
from jax.experimental import pallas as pl
from jax.experimental.pallas import tpu as pltpu
from jax.experimental.pallas import tpu_sc as plsc

_N = 10000
_D = 128
_E = 320000
_NC = 2
_NS = 16
_NW = _NC * _NS
_EPT = _E // _NW
_K = 80
_CH = _EPT // _K
_NPT = 632
_NPL = _N - (_NS - 1) * _NPT
_BM = 400


def _gemm_body(x_ref, w_ref, o_ref):
    o_ref[...] = lax.dot_general(
        x_ref[...], w_ref[...], (((1,), (1,)), ((), ())),
        preferred_element_type=jnp.float32)


def _add_body(a_ref, b_ref, o_ref):
    o_ref[...] = a_ref[...] + b_ref[...]


def _seg_body(src_hbm, dst_hbm, h_hbm, z_hbm, out_hbm,
              src_idx, dst_idx, rows, sems, acc):
    c = lax.axis_index("c")
    s = lax.axis_index("s")
    w = c * _NS + s
    pltpu.sync_copy(src_hbm.at[w], src_idx)
    pltpu.sync_copy(dst_hbm.at[w], dst_idx)
    @pl.when(s < _NS - 1)
    def _():
        pltpu.sync_copy(z_hbm.at[pl.ds(s * _NPT, _NPT)],
                        acc.at[pl.ds(s * _NPT, _NPT)])

    @pl.when(s == _NS - 1)
    def _():
        pltpu.sync_copy(z_hbm.at[pl.ds(s * _NPT, _NPL)],
                        acc.at[pl.ds(s * _NPT, _NPL)])

    plsc.subcore_barrier()

    def fire(j, b):
        pltpu.async_copy(h_hbm.at[src_idx.at[pl.ds(j * _K, _K)]],
                         rows.at[b], sems.at[b])

    fire(0, 0)
    fire(1, 1)

    def chunk(j, carry):
        b = lax.rem(j, 2)
        pltpu.make_async_copy(h_hbm.at[pl.ds(0, _K)], rows.at[b],
                              sems.at[b]).wait()
        pltpu.sync_copy(rows.at[b], acc.at[dst_idx.at[j]], add=True)

        @pl.when(j + 2 < _CH)
        def _():
            fire(j + 2, b)
        return carry

    lax.fori_loop(0, _CH, chunk, 0)

    plsc.subcore_barrier()

    @pl.when(s < _NS - 1)
    def _():
        pltpu.sync_copy(acc.at[pl.ds(s * _NPT, _NPT)],
                        out_hbm.at[c, pl.ds(s * _NPT, _NPT)])

    @pl.when(s == _NS - 1)
    def _():
        pltpu.sync_copy(acc.at[pl.ds(s * _NPT, _NPL)],
                        out_hbm.at[c, pl.ds(s * _NPT, _NPL)])


def kernel(x, edge_index, weight):
    n, d = x.shape

    h = pl.pallas_call(
        _gemm_body,
        grid=(n // _BM,),
        in_specs=[pl.BlockSpec((_BM, d), lambda i: (i, 0)),
                  pl.BlockSpec(weight.shape, lambda i: (0, 0))],
        out_specs=pl.BlockSpec((_BM, d), lambda i: (i, 0)),
        out_shape=jax.ShapeDtypeStruct((n, d), jnp.float32),
    )(x, weight)

    src = edge_index[0].reshape(_NW, _EPT)
    dst = edge_index[1].reshape(_NW, _CH, _K)
    zeros = jnp.zeros((n, d), jnp.float32)

    mesh = plsc.VectorSubcoreMesh(core_axis_name="c", subcore_axis_name="s")
    seg = pl.kernel(
        _seg_body,
        out_type=jax.ShapeDtypeStruct((_NC, n, d), jnp.float32),
        mesh=mesh,
        scratch_types=[
            pltpu.VMEM((_EPT,), jnp.int32),
            pltpu.VMEM((_CH, _K), jnp.int32),
            pltpu.VMEM((2, _K, _D), jnp.float32),
            pltpu.SemaphoreType.DMA((2,)),
            pltpu.VMEM_SHARED((_N, _D), jnp.float32),
        ],
    )
    parts = seg(src, dst, h, zeros)

    out = pl.pallas_call(
        _add_body,
        grid=(n // _BM,),
        in_specs=[pl.BlockSpec((_BM, d), lambda i: (i, 0)),
                  pl.BlockSpec((_BM, d), lambda i: (i, 0))],
        out_specs=pl.BlockSpec((_BM, d), lambda i: (i, 0)),
        out_shape=jax.ShapeDtypeStruct((n, d), jnp.float32),
    )(parts[0], parts[1])
    return out
